# Initial kernel scaffold; baseline (speedup 1.0000x reference)
import jax
import jax.numpy as jnp
from jax.experimental import pallas as pl


def kernel(inputs, tok_table, pos_table):
    raise NotImplementedError("write your pallas kernel here")


# speedup vs baseline: 1.0226x; 1.0226x over previous
import functools

import jax
import jax.numpy as jnp
from jax import lax
from jax.experimental import pallas as pl
from jax.experimental.pallas import tpu as pltpu
from jax.experimental.pallas import tpu_sc as plsc

D_MODEL = 768
SEQ_LEN = 2048
LANES = 16
COLS = D_MODEL // LANES


def _sc_embed(idx_flat, tok_table, pos_table):
    n_tok = idx_flat.shape[0]
    info = plsc.get_sparse_core_info()
    nc, ns = info.num_cores, info.num_subcores
    nw = nc * ns
    tok_per_w = n_tok // nw
    ch = 64
    n_ch = tok_per_w // ch

    mesh = plsc.VectorSubcoreMesh(core_axis_name="c", subcore_axis_name="s")

    @functools.partial(
        pl.kernel,
        mesh=mesh,
        out_type=jax.ShapeDtypeStruct((n_tok, D_MODEL), jnp.float32),
        scratch_types=[
            pltpu.VMEM((tok_per_w,), jnp.int32),
            pltpu.VMEM((ch, D_MODEL), jnp.float32),
            pltpu.VMEM((ch, D_MODEL), jnp.float32),
            pltpu.SemaphoreType.DMA,
        ],
    )
    def k(idx_hbm, tok_hbm, pos_hbm, out_hbm, idx_v, rows_v, pos_v, sem):
        wid = lax.axis_index("s") * nc + lax.axis_index("c")
        base = wid * tok_per_w
        pltpu.sync_copy(idx_hbm.at[pl.ds(base, tok_per_w)], idx_v)
        for c in range(n_ch):
            cbase = base + c * ch
            pos_base = lax.rem(cbase, SEQ_LEN)
            gat = pltpu.async_copy(
                tok_hbm.at[idx_v.at[pl.ds(c * ch, ch)]], rows_v, sem)
            pltpu.sync_copy(pos_hbm.at[pl.ds(pos_base, ch)], pos_v)
            gat.wait()

            def row_body(r, carry):
                for cc in range(COLS):
                    x = pos_v[r, pl.ds(cc * LANES, LANES)]
                    plsc.addupdate(rows_v.at[r, pl.ds(cc * LANES, LANES)], x)
                return carry

            lax.fori_loop(0, ch, row_body, 0)
            pltpu.sync_copy(rows_v, out_hbm.at[pl.ds(cbase, ch)])

    return k(idx_flat, tok_table, pos_table)


def kernel(inputs, tok_table, pos_table):
    b, l = inputs.shape
    out = _sc_embed(inputs.reshape(b * l), tok_table, pos_table)
    return out.reshape(b, l, D_MODEL)


# speedup vs baseline: 1.0481x; 1.0249x over previous
import functools

import jax
import jax.numpy as jnp
from jax import lax
from jax.experimental import pallas as pl
from jax.experimental.pallas import tpu as pltpu
from jax.experimental.pallas import tpu_sc as plsc

D_MODEL = 768
LANES = 16


def _sc_embed(idx_wmajor, tok_table, pos_table, batch, seq_len):
    n_tok = batch * seq_len
    info = plsc.get_sparse_core_info()
    nc, ns = info.num_cores, info.num_subcores
    nw = nc * ns
    pos_per_w = seq_len // nw
    ch = 32
    hsub = pos_per_w // ch
    n_ch = batch * hsub

    mesh = plsc.VectorSubcoreMesh(core_axis_name="c", subcore_axis_name="s")

    @functools.partial(
        pl.kernel,
        mesh=mesh,
        out_type=jax.ShapeDtypeStruct((n_tok, D_MODEL), jnp.float32),
        scratch_types=[
            pltpu.VMEM((n_ch, ch), jnp.int32),
            pltpu.VMEM((pos_per_w, D_MODEL), jnp.float32),
            pltpu.VMEM((ch, D_MODEL), jnp.float32),
            pltpu.VMEM((ch, D_MODEL), jnp.float32),
            pltpu.SemaphoreType.DMA,
            pltpu.SemaphoreType.DMA,
            pltpu.SemaphoreType.DMA,
            pltpu.SemaphoreType.DMA,
        ],
    )
    def k(idx_hbm, tok_hbm, pos_hbm, out_hbm,
          idx_v, pos_v, buf_a, buf_b, g0, g1, o0, o1):
        wid = lax.axis_index("s") * nc + lax.axis_index("c")
        pltpu.sync_copy(idx_hbm.at[wid], idx_v)
        pltpu.sync_copy(pos_hbm.at[pl.ds(wid * pos_per_w, pos_per_w)], pos_v)

        bufs = (buf_a, buf_b)
        gsems = (g0, g1)
        osems = (o0, o1)

        def out_base(r):
            b, h = divmod(r, hsub)
            return b * seq_len + wid * pos_per_w + h * ch

        gather0 = pltpu.async_copy(tok_hbm.at[idx_v.at[0]], bufs[0], gsems[0])
        out_copies = [None, None]
        for r in range(n_ch):
            cur = r & 1
            nxt = 1 - cur
            if r + 1 < n_ch:
                if out_copies[nxt] is not None:
                    out_copies[nxt].wait()
                next_gather = pltpu.async_copy(
                    tok_hbm.at[idx_v.at[r + 1]], bufs[nxt], gsems[nxt])
            if r == 0:
                gather0.wait()
            else:
                cur_gather.wait()
            h = r % hsub
            buf = bufs[cur]

            def row_body(row, carry, _h=h, _buf=buf):
                for cc in range(D_MODEL // LANES):
                    x = pos_v[_h * ch + row, pl.ds(cc * LANES, LANES)]
                    plsc.addupdate(_buf.at[row, pl.ds(cc * LANES, LANES)], x)
                return carry

            lax.fori_loop(0, ch, row_body, 0)
            out_copies[cur] = pltpu.async_copy(
                bufs[cur], out_hbm.at[pl.ds(out_base(r), ch)], osems[cur])
            if r + 1 < n_ch:
                cur_gather = next_gather
        out_copies[0].wait()
        out_copies[1].wait()

    return k(idx_wmajor, tok_table, pos_table)


def kernel(inputs, tok_table, pos_table):
    b, l = inputs.shape
    nw = 32
    pos_per_w = l // nw
    ch = 32
    idx_wmajor = (inputs.reshape(b, nw, pos_per_w // ch, ch)
                  .transpose(1, 0, 2, 3)
                  .reshape(nw, b * (pos_per_w // ch), ch))
    out = _sc_embed(idx_wmajor, tok_table, pos_table, b, l)
    return out.reshape(b, l, D_MODEL)
